# baseline (speedup 1.0000x reference)
import functools

import jax
import jax.numpy as jnp
from jax import lax
from jax.experimental import pallas as pl
from jax.experimental.pallas import tpu as pltpu
from jax.experimental.pallas import tpu_sc as plsc

N = 10000
F = 128
H = 64
B = 64
OUT = 3
EPS = 1e-3
BLK = 400
HBLK = BLK // 2
NBLK = N // BLK
LANE = 16


def _layernorm(h, gamma, beta):
    mu = jnp.mean(h, axis=-1, keepdims=True)
    d = h - mu
    var = jnp.mean(d * d, axis=-1, keepdims=True)
    return d * lax.rsqrt(var + EPS) * gamma + beta


def _starts_body(icol_ref, starts_ref):
    icol = icol_ref[...]
    bl = lax.broadcasted_iota(jnp.int32, (N, 128), 1)
    lt = (icol > bl).astype(jnp.float32)
    starts_ref[...] = jnp.sum(lt, axis=0, keepdims=True).astype(jnp.int32)


def _starts(icol):
    return pl.pallas_call(
        _starts_body,
        out_shape=jax.ShapeDtypeStruct((1, 128), jnp.int32),
    )(icol)


def _main_body(x_ref, w1_ref, a0_ref, a1_ref, b1_ref, g1_ref, be1_ref,
               w2_ref, b2_ref, g2_ref, be2_ref, wf_ref, bf_ref, wa_ref,
               ba_ref, i_ref, g_ref, xw_s, h1_s, rhs2_s):
    m = pl.program_id(0)
    mm = m % NBLK

    @pl.when(m == 0)
    def _():
        xw_s[...] = jnp.dot(x_ref[...], w1_ref[...],
                            preferred_element_type=jnp.float32)
        g_ref[...] = jnp.zeros_like(g_ref)

    @pl.when(m < NBLK)
    def _():
        for half, a_ref in enumerate((a0_ref, a1_ref)):
            rows = pl.ds(mm * BLK + half * HBLK, HBLK)
            t = jnp.dot(a_ref[...], xw_s[...],
                        preferred_element_type=jnp.float32)
            h = jnp.maximum(t + b1_ref[...], 0.0)
            hn = _layernorm(h, g1_ref[...], be1_ref[...])
            h1_s[rows, :] = hn
            rhs2_s[rows, :] = jnp.dot(hn, w2_ref[...],
                                      preferred_element_type=jnp.float32)

    @pl.when(m >= NBLK)
    def _():
        part = jnp.zeros((B, H), jnp.float32)
        for half, a_ref in enumerate((a0_ref, a1_ref)):
            rows = pl.ds(mm * BLK + half * HBLK, HBLK)
            t = jnp.dot(a_ref[...], rhs2_s[...],
                        preferred_element_type=jnp.float32)
            h = jnp.maximum(t + b2_ref[...], 0.0)
            h = _layernorm(h, g2_ref[...], be2_ref[...]) + h1_s[rows, :]
            feat = jnp.dot(h, wf_ref[...],
                           preferred_element_type=jnp.float32) + bf_ref[...]
            attn = jax.nn.sigmoid(
                jnp.dot(h, wa_ref[...], preferred_element_type=jnp.float32)
                + ba_ref[...])
            contrib = feat * attn
            seg = i_ref[0, half]
            onehot = (lax.broadcasted_iota(jnp.int32, (B, HBLK), 0) == seg)
            part = part + jnp.dot(onehot.astype(jnp.float32), contrib,
                                  preferred_element_type=jnp.float32)
        g_ref[...] += part


def _main(x, w1, a, b1, g1, be1, w2, b2, g2, be2, wf, bf, wa, ba, i3d):
    vec = lambda: pl.BlockSpec((1, H), lambda m: (0, 0))
    mat = lambda: pl.BlockSpec((H, H), lambda m: (0, 0))
    return pl.pallas_call(
        _main_body,
        grid=(2 * NBLK,),
        in_specs=[
            pl.BlockSpec((N, F), lambda m: (0, 0)),
            pl.BlockSpec((F, H), lambda m: (0, 0)),
            pl.BlockSpec((HBLK, N), lambda m: (2 * (m % NBLK), 0)),
            pl.BlockSpec((HBLK, N), lambda m: (2 * (m % NBLK) + 1, 0)),
            vec(), vec(), vec(),
            mat(),
            vec(), vec(), vec(),
            mat(), vec(), mat(), vec(),
            pl.BlockSpec((1, 2, HBLK), lambda m: (m % NBLK, 0, 0)),
        ],
        out_specs=pl.BlockSpec((B, H), lambda m: (0, 0)),
        out_shape=jax.ShapeDtypeStruct((B, H), jnp.float32),
        scratch_shapes=[
            pltpu.VMEM((N, H), jnp.float32),
            pltpu.VMEM((N, H), jnp.float32),
            pltpu.VMEM((N, H), jnp.float32),
        ],
        compiler_params=pltpu.CompilerParams(
            dimension_semantics=("arbitrary",)),
    )(x, w1, a, a, b1, g1, be1, w2, b2, g2, be2, wf, bf, wa, ba, i3d)


def _bary_sc(sig, xs, ys, zs, starts_exp):
    mesh = plsc.VectorSubcoreMesh(core_axis_name="c", subcore_axis_name="s")

    @functools.partial(
        pl.kernel,
        mesh=mesh,
        out_type=jax.ShapeDtypeStruct((B, LANE), jnp.float32),
        scratch_types=[
            pltpu.VMEM((N,), jnp.float32),
            pltpu.VMEM((N,), jnp.float32),
            pltpu.VMEM((N,), jnp.float32),
            pltpu.VMEM((N,), jnp.float32),
            pltpu.VMEM((LANE,), jnp.int32),
            pltpu.VMEM((2, LANE), jnp.float32),
        ],
    )
    def bary_kernel(sig_hbm, xs_hbm, ys_hbm, zs_hbm, se_hbm, out_hbm,
                    sig_v, xs_v, ys_v, zs_v, st_v, out_v):
        wid = lax.axis_index("s") * 2 + lax.axis_index("c")
        pltpu.sync_copy(sig_hbm, sig_v)
        pltpu.sync_copy(xs_hbm, xs_v)
        pltpu.sync_copy(ys_hbm, ys_v)
        pltpu.sync_copy(zs_hbm, zs_v)
        b0 = wid * 2
        pltpu.sync_copy(se_hbm.at[pl.ds(wid * 8, 8)], st_v.at[pl.ds(0, 8)])
        sv = st_v[...]
        s0 = sv[0]
        s1 = sv[1]
        s2 = sv[2]
        il = lax.broadcasted_iota(jnp.int32, (LANE,), 0)

        def allsum(v):
            dnums = lax.GatherDimensionNumbers(
                offset_dims=(), collapsed_slice_dims=(0,),
                start_index_map=(0,))
            for k in (8, 4, 2, 1):
                perm = lax.gather(
                    v, (il ^ k)[:, None], dnums, slice_sizes=(1,),
                    mode=lax.GatherScatterMode.PROMISE_IN_BOUNDS)
                v = v + perm
            return v

        def seg_sums(start, end):
            c0 = start // LANE
            c1 = (end + LANE - 1) // LANE

            def body(c, acc):
                accs, accx, accy, accz = acc
                off = c * LANE
                lane = off + il
                msk = (lane >= start) & (lane < end)
                sg = jnp.maximum(sig_v[pl.ds(off, LANE)], 0.0)
                w = jnp.where(msk, 1.0 + sg, 0.0)
                return (accs + w,
                        accx + w * xs_v[pl.ds(off, LANE)],
                        accy + w * ys_v[pl.ds(off, LANE)],
                        accz + w * zs_v[pl.ds(off, LANE)])

            z16 = jnp.zeros((LANE,), jnp.float32)
            accs, accx, accy, accz = lax.fori_loop(
                c0, c1, body, (z16, z16, z16, z16))
            return (allsum(accs), allsum(accx),
                    allsum(accy), allsum(accz))

        for j, (lo, hi) in enumerate(((s0, s1), (s1, s2))):
            ssum, ex, ey, ez = seg_sums(lo, hi)
            row = jnp.where(
                il == 0, ssum,
                jnp.where(il == 1, ex,
                          jnp.where(il == 2, ey,
                                    jnp.where(il == 3, ez,
                                              jnp.zeros((LANE,),
                                                        jnp.float32)))))
            out_v[j] = row
        pltpu.sync_copy(out_v, out_hbm.at[pl.ds(b0, 2)])

    return bary_kernel(sig, xs, ys, zs, starts_exp)


def _head_body(g_ref, sums_ref, woh_ref, wob_ref, bo_ref, out_ref):
    sums = sums_ref[...]
    s = sums[:, 0:1]
    e = sums[:, 1:4]
    bary = jnp.where(s > 0, e / jnp.where(s > 0, s, 1.0), 0.0)
    out_ref[...] = (
        jnp.dot(g_ref[...], woh_ref[...], preferred_element_type=jnp.float32)
        + jnp.dot(bary, wob_ref[...], preferred_element_type=jnp.float32)
        + bo_ref[...])


def _head(g, sums, woh, wob, bo):
    return pl.pallas_call(
        _head_body,
        out_shape=jax.ShapeDtypeStruct((B, OUT), jnp.float32),
    )(g, sums, woh, wob, bo)


def kernel(x, a, i, W1, b1, W2, b2, g1, be1, g2, be2, Wf, bf, Wa, ba, Wo, bo):
    seg = i.astype(jnp.int32)
    icol = seg.reshape(N, 1)
    i3d = seg.reshape(NBLK, 2, HBLK)
    row = lambda v: v.reshape(1, H)

    starts2d = _starts(icol)
    starts = starts2d.reshape(128)
    eidx = 2 * jnp.arange(32, dtype=jnp.int32)[:, None] \
        + jnp.arange(8, dtype=jnp.int32)[None, :]
    starts_exp = starts[eidx].reshape(256)

    sums = _bary_sc(x[:, 0], x[:, F - 3], x[:, F - 2], x[:, F - 1],
                    starts_exp)

    g = _main(x, W1, a, row(b1), row(g1), row(be1), W2,
              row(b2), row(g2), row(be2), Wf, row(bf), Wa, row(ba), i3d)

    return _head(g, sums, Wo[:H, :], Wo[H:, :], bo.reshape(1, OUT))


# speedup vs baseline: 1.1190x; 1.1190x over previous
import functools

import jax
import jax.numpy as jnp
from jax import lax
from jax.experimental import pallas as pl
from jax.experimental.pallas import tpu as pltpu
from jax.experimental.pallas import tpu_sc as plsc

N = 10000
F = 128
H = 64
B = 64
OUT = 3
EPS = 1e-3
BLK = 400
HBLK = BLK // 2
NBLK = N // BLK
LANE = 16


def _layernorm(h, gamma, beta):
    mu = jnp.mean(h, axis=-1, keepdims=True)
    d = h - mu
    var = jnp.mean(d * d, axis=-1, keepdims=True)
    return d * lax.rsqrt(var + EPS) * gamma + beta


def _starts_body(icol_ref, starts_ref):
    icol = icol_ref[...]
    bl = lax.broadcasted_iota(jnp.int32, (N, 128), 1)
    lt = (icol > bl).astype(jnp.float32)
    starts_ref[...] = jnp.sum(lt, axis=0, keepdims=True).astype(jnp.int32)


def _starts(icol):
    return pl.pallas_call(
        _starts_body,
        out_shape=jax.ShapeDtypeStruct((1, 128), jnp.int32),
    )(icol)


def _main_body(x_ref, w1_ref, a0_ref, b1_ref, g1_ref, be1_ref,
               w2_ref, b2_ref, g2_ref, be2_ref, wf_ref, bf_ref, wa_ref,
               ba_ref, i_ref, g_ref, xw_s, h1_s, rhs2_s):
    m = pl.program_id(0)
    mm = m % NBLK

    @pl.when(m == 0)
    def _():
        xw_s[...] = jnp.dot(x_ref[...], w1_ref[...],
                            preferred_element_type=jnp.float32)
        g_ref[...] = jnp.zeros_like(g_ref)

    @pl.when(m < NBLK)
    def _():
        rows = pl.ds(mm * BLK, BLK)
        t = jnp.dot(a0_ref[...], xw_s[...],
                    preferred_element_type=jnp.float32)
        h = jnp.maximum(t + b1_ref[...], 0.0)
        hn = _layernorm(h, g1_ref[...], be1_ref[...])
        h1_s[rows, :] = hn
        rhs2_s[rows, :] = jnp.dot(hn, w2_ref[...],
                                  preferred_element_type=jnp.float32)

    @pl.when(m >= NBLK)
    def _():
        rows = pl.ds(mm * BLK, BLK)
        t = jnp.dot(a0_ref[...], rhs2_s[...],
                    preferred_element_type=jnp.float32)
        h = jnp.maximum(t + b2_ref[...], 0.0)
        h = _layernorm(h, g2_ref[...], be2_ref[...]) + h1_s[rows, :]
        feat = jnp.dot(h, wf_ref[...],
                       preferred_element_type=jnp.float32) + bf_ref[...]
        attn = jax.nn.sigmoid(
            jnp.dot(h, wa_ref[...], preferred_element_type=jnp.float32)
            + ba_ref[...])
        contrib = feat * attn
        seg = i_ref[0]
        onehot = (lax.broadcasted_iota(jnp.int32, (B, BLK), 0) == seg)
        part = jnp.dot(onehot.astype(jnp.float32), contrib,
                       preferred_element_type=jnp.float32)
        g_ref[...] += part


def _main(x, w1, a, b1, g1, be1, w2, b2, g2, be2, wf, bf, wa, ba, i3d):
    vec = lambda: pl.BlockSpec((1, H), lambda m: (0, 0))
    mat = lambda: pl.BlockSpec((H, H), lambda m: (0, 0))
    return pl.pallas_call(
        _main_body,
        grid=(2 * NBLK,),
        in_specs=[
            pl.BlockSpec((N, F), lambda m: (0, 0)),
            pl.BlockSpec((F, H), lambda m: (0, 0)),
            pl.BlockSpec((BLK, N), lambda m: (m % NBLK, 0)),
            vec(), vec(), vec(),
            mat(),
            vec(), vec(), vec(),
            mat(), vec(), mat(), vec(),
            pl.BlockSpec((1, 1, BLK), lambda m: (m % NBLK, 0, 0)),
        ],
        out_specs=pl.BlockSpec((B, H), lambda m: (0, 0)),
        out_shape=jax.ShapeDtypeStruct((B, H), jnp.float32),
        scratch_shapes=[
            pltpu.VMEM((N, H), jnp.float32),
            pltpu.VMEM((N, H), jnp.float32),
            pltpu.VMEM((N, H), jnp.float32),
        ],
        compiler_params=pltpu.CompilerParams(
            dimension_semantics=("arbitrary",)),
    )(x, w1, a, b1, g1, be1, w2, b2, g2, be2, wf, bf, wa, ba, i3d)


def _bary_sc(xt4, starts_exp):
    mesh = plsc.VectorSubcoreMesh(core_axis_name="c", subcore_axis_name="s")

    @functools.partial(
        pl.kernel,
        mesh=mesh,
        out_type=jax.ShapeDtypeStruct((B, LANE), jnp.float32),
        scratch_types=[
            pltpu.VMEM((4, N), jnp.float32),
            pltpu.VMEM((LANE,), jnp.int32),
            pltpu.VMEM((2, LANE), jnp.float32),
        ],
    )
    def bary_kernel(xt4_hbm, se_hbm, out_hbm, xt_v, st_v, out_v):
        wid = lax.axis_index("s") * 2 + lax.axis_index("c")
        pltpu.sync_copy(xt4_hbm, xt_v)
        sig_v = xt_v.at[0]
        xs_v = xt_v.at[1]
        ys_v = xt_v.at[2]
        zs_v = xt_v.at[3]
        b0 = wid * 2
        pltpu.sync_copy(se_hbm.at[pl.ds(wid * 8, 8)], st_v.at[pl.ds(0, 8)])
        sv = st_v[...]
        s0 = sv[0]
        s1 = sv[1]
        s2 = sv[2]
        il = lax.broadcasted_iota(jnp.int32, (LANE,), 0)

        def allsum(v):
            dnums = lax.GatherDimensionNumbers(
                offset_dims=(), collapsed_slice_dims=(0,),
                start_index_map=(0,))
            for k in (8, 4, 2, 1):
                perm = lax.gather(
                    v, (il ^ k)[:, None], dnums, slice_sizes=(1,),
                    mode=lax.GatherScatterMode.PROMISE_IN_BOUNDS)
                v = v + perm
            return v

        def seg_sums(start, end):
            c0 = start // LANE
            c1 = (end + LANE - 1) // LANE

            def body(c, acc):
                accs, accx, accy, accz = acc
                off = c * LANE
                lane = off + il
                msk = (lane >= start) & (lane < end)
                sg = jnp.maximum(sig_v[pl.ds(off, LANE)], 0.0)
                w = jnp.where(msk, 1.0 + sg, 0.0)
                return (accs + w,
                        accx + w * xs_v[pl.ds(off, LANE)],
                        accy + w * ys_v[pl.ds(off, LANE)],
                        accz + w * zs_v[pl.ds(off, LANE)])

            z16 = jnp.zeros((LANE,), jnp.float32)
            accs, accx, accy, accz = lax.fori_loop(
                c0, c1, body, (z16, z16, z16, z16))
            return (allsum(accs), allsum(accx),
                    allsum(accy), allsum(accz))

        for j, (lo, hi) in enumerate(((s0, s1), (s1, s2))):
            ssum, ex, ey, ez = seg_sums(lo, hi)
            row = jnp.where(
                il == 0, ssum,
                jnp.where(il == 1, ex,
                          jnp.where(il == 2, ey,
                                    jnp.where(il == 3, ez,
                                              jnp.zeros((LANE,),
                                                        jnp.float32)))))
            out_v[j] = row
        pltpu.sync_copy(out_v, out_hbm.at[pl.ds(b0, 2)])

    return bary_kernel(xt4, starts_exp)


def _head_body(g_ref, sums_ref, woh_ref, wob_ref, bo_ref, out_ref):
    sums = sums_ref[...]
    s = sums[:, 0:1]
    e = sums[:, 1:4]
    bary = jnp.where(s > 0, e / jnp.where(s > 0, s, 1.0), 0.0)
    out_ref[...] = (
        jnp.dot(g_ref[...], woh_ref[...], preferred_element_type=jnp.float32)
        + jnp.dot(bary, wob_ref[...], preferred_element_type=jnp.float32)
        + bo_ref[...])


def _head(g, sums, woh, wob, bo):
    return pl.pallas_call(
        _head_body,
        out_shape=jax.ShapeDtypeStruct((B, OUT), jnp.float32),
    )(g, sums, woh, wob, bo)


def kernel(x, a, i, W1, b1, W2, b2, g1, be1, g2, be2, Wf, bf, Wa, ba, Wo, bo):
    seg = i.astype(jnp.int32)
    icol = seg.reshape(N, 1)
    i3d = seg.reshape(NBLK, 1, BLK)
    row = lambda v: v.reshape(1, H)

    starts2d = _starts(icol)
    starts = starts2d.reshape(128)
    eidx = 2 * jnp.arange(32, dtype=jnp.int32)[:, None] \
        + jnp.arange(8, dtype=jnp.int32)[None, :]
    starts_exp = starts[eidx].reshape(256)

    xt4 = jnp.stack([x[:, 0], x[:, F - 3], x[:, F - 2], x[:, F - 1]])
    sums = _bary_sc(xt4, starts_exp)

    g = _main(x, W1, a, row(b1), row(g1), row(be1), W2,
              row(b2), row(g2), row(be2), Wf, row(bf), Wa, row(ba), i3d)

    return _head(g, sums, Wo[:H, :], Wo[H:, :], bo.reshape(1, OUT))


# speedup vs baseline: 1.1893x; 1.0628x over previous
import functools

import jax
import jax.numpy as jnp
from jax import lax
from jax.experimental import pallas as pl
from jax.experimental.pallas import tpu as pltpu
from jax.experimental.pallas import tpu_sc as plsc

N = 10000
F = 128
H = 64
B = 64
OUT = 3
EPS = 1e-3
BLK = 400
HBLK = BLK // 2
NBLK = N // BLK
LANE = 16


def _layernorm(h, gamma, beta):
    mu = jnp.mean(h, axis=-1, keepdims=True)
    d = h - mu
    var = jnp.mean(d * d, axis=-1, keepdims=True)
    return d * lax.rsqrt(var + EPS) * gamma + beta


def _xt4_body(x_ref, xt4_ref):
    r = lax.broadcasted_iota(jnp.int32, (8, F), 0)
    f = lax.broadcasted_iota(jnp.int32, (8, F), 1)
    tgt = jnp.where(r == 0, 0,
                    jnp.where(r == 1, F - 3,
                              jnp.where(r == 2, F - 2,
                                        jnp.where(r == 3, F - 1, -1))))
    sel = (f == tgt).astype(jnp.float32)
    xt4_ref[...] = lax.dot_general(
        sel, x_ref[...], (((1,), (1,)), ((), ())),
        preferred_element_type=jnp.float32)


def _xt4(x):
    return pl.pallas_call(
        _xt4_body,
        out_shape=jax.ShapeDtypeStruct((8, N), jnp.float32),
    )(x)


def _main_body(x_ref, w1_ref, a0_ref, b1_ref, g1_ref, be1_ref,
               w2_ref, b2_ref, g2_ref, be2_ref, wf_ref, bf_ref, wa_ref,
               ba_ref, i_ref, g_ref, xw_s, h1_s, rhs2_s):
    m = pl.program_id(0)
    mm = m % NBLK

    @pl.when(m == 0)
    def _():
        xw_s[...] = jnp.dot(x_ref[...], w1_ref[...],
                            preferred_element_type=jnp.float32)
        g_ref[...] = jnp.zeros_like(g_ref)

    @pl.when(m < NBLK)
    def _():
        rows = pl.ds(mm * BLK, BLK)
        t = jnp.dot(a0_ref[...], xw_s[...],
                    preferred_element_type=jnp.float32)
        h = jnp.maximum(t + b1_ref[...], 0.0)
        hn = _layernorm(h, g1_ref[...], be1_ref[...])
        h1_s[rows, :] = hn
        rhs2_s[rows, :] = jnp.dot(hn, w2_ref[...],
                                  preferred_element_type=jnp.float32)

    @pl.when(m >= NBLK)
    def _():
        rows = pl.ds(mm * BLK, BLK)
        t = jnp.dot(a0_ref[...], rhs2_s[...],
                    preferred_element_type=jnp.float32)
        h = jnp.maximum(t + b2_ref[...], 0.0)
        h = _layernorm(h, g2_ref[...], be2_ref[...]) + h1_s[rows, :]
        feat = jnp.dot(h, wf_ref[...],
                       preferred_element_type=jnp.float32) + bf_ref[...]
        attn = jax.nn.sigmoid(
            jnp.dot(h, wa_ref[...], preferred_element_type=jnp.float32)
            + ba_ref[...])
        contrib = feat * attn
        seg = i_ref[0]
        onehot = (lax.broadcasted_iota(jnp.int32, (B, BLK), 0) == seg)
        part = jnp.dot(onehot.astype(jnp.float32), contrib,
                       preferred_element_type=jnp.float32)
        g_ref[...] += part


def _main(x, w1, a, b1, g1, be1, w2, b2, g2, be2, wf, bf, wa, ba, i3d):
    vec = lambda: pl.BlockSpec((1, H), lambda m: (0, 0))
    mat = lambda: pl.BlockSpec((H, H), lambda m: (0, 0))
    return pl.pallas_call(
        _main_body,
        grid=(2 * NBLK,),
        in_specs=[
            pl.BlockSpec((N, F), lambda m: (0, 0)),
            pl.BlockSpec((F, H), lambda m: (0, 0)),
            pl.BlockSpec((BLK, N), lambda m: (m % NBLK, 0)),
            vec(), vec(), vec(),
            mat(),
            vec(), vec(), vec(),
            mat(), vec(), mat(), vec(),
            pl.BlockSpec((1, 1, BLK), lambda m: (m % NBLK, 0, 0)),
        ],
        out_specs=pl.BlockSpec((B, H), lambda m: (0, 0)),
        out_shape=jax.ShapeDtypeStruct((B, H), jnp.float32),
        scratch_shapes=[
            pltpu.VMEM((N, H), jnp.float32),
            pltpu.VMEM((N, H), jnp.float32),
            pltpu.VMEM((N, H), jnp.float32),
        ],
        compiler_params=pltpu.CompilerParams(
            dimension_semantics=("arbitrary",)),
    )(x, w1, a, b1, g1, be1, w2, b2, g2, be2, wf, bf, wa, ba, i3d)


def _bary_sc(xt4, iseg):
    mesh = plsc.VectorSubcoreMesh(core_axis_name="c", subcore_axis_name="s")

    @functools.partial(
        pl.kernel,
        mesh=mesh,
        out_type=jax.ShapeDtypeStruct((B, LANE), jnp.float32),
        scratch_types=[
            pltpu.VMEM((4, N), jnp.float32),
            pltpu.VMEM((N,), jnp.int32),
            pltpu.VMEM((2, LANE), jnp.float32),
        ],
    )
    def bary_kernel(xt4_hbm, iseg_hbm, out_hbm, xt_v, iseg_v, out_v):
        wid = lax.axis_index("s") * 2 + lax.axis_index("c")
        pltpu.sync_copy(xt4_hbm.at[pl.ds(0, 4)], xt_v)
        pltpu.sync_copy(iseg_hbm, iseg_v)
        sig_v = xt_v.at[0]
        xs_v = xt_v.at[1]
        ys_v = xt_v.at[2]
        zs_v = xt_v.at[3]
        b0 = wid * 2
        il = lax.broadcasted_iota(jnp.int32, (LANE,), 0)
        dnums = lax.GatherDimensionNumbers(
            offset_dims=(), collapsed_slice_dims=(0,),
            start_index_map=(0,))

        NCH = N // LANE

        def allsum(v):
            for k in (8, 4, 2, 1):
                perm = lax.gather(
                    v, (il ^ k)[:, None], dnums, slice_sizes=(1,),
                    mode=lax.GatherScatterMode.PROMISE_IN_BOUNDS)
                v = v + perm
            return v

        def find_ge(b):
            def step(_, c):
                lo, hi = c
                mid = lax.div(lo + hi, 2)
                v = iseg_v[pl.ds(mid * LANE, LANE)]
                lt = v[LANE - 1] < b
                return (jnp.where(lt, mid + 1, lo), jnp.where(lt, hi, mid))

            ch = lax.fori_loop(0, 10, step, (0, NCH))[0]
            ch = jnp.minimum(ch, NCH - 1)
            v = iseg_v[pl.ds(ch * LANE, LANE)]
            cnt = allsum(jnp.where(v < b, 1.0, 0.0))[0].astype(jnp.int32)
            return ch * LANE + cnt

        s0 = find_ge(b0)
        s1 = find_ge(b0 + 1)
        s2 = find_ge(b0 + 2)

        def seg_sums(start, end):
            c0 = start // LANE
            c1 = (end + LANE - 1) // LANE

            def body(c, acc):
                accs, accx, accy, accz = acc
                off = c * LANE
                lane = off + il
                msk = (lane >= start) & (lane < end)
                sg = jnp.maximum(sig_v[pl.ds(off, LANE)], 0.0)
                w = jnp.where(msk, 1.0 + sg, 0.0)
                return (accs + w,
                        accx + w * xs_v[pl.ds(off, LANE)],
                        accy + w * ys_v[pl.ds(off, LANE)],
                        accz + w * zs_v[pl.ds(off, LANE)])

            z16 = jnp.zeros((LANE,), jnp.float32)
            accs, accx, accy, accz = lax.fori_loop(
                c0, c1, body, (z16, z16, z16, z16))
            return (allsum(accs), allsum(accx),
                    allsum(accy), allsum(accz))

        for j, (lo, hi) in enumerate(((s0, s1), (s1, s2))):
            ssum, ex, ey, ez = seg_sums(lo, hi)
            row = jnp.where(
                il == 0, ssum,
                jnp.where(il == 1, ex,
                          jnp.where(il == 2, ey,
                                    jnp.where(il == 3, ez,
                                              jnp.zeros((LANE,),
                                                        jnp.float32)))))
            out_v[j] = row
        pltpu.sync_copy(out_v, out_hbm.at[pl.ds(b0, 2)])

    return bary_kernel(xt4, iseg)


def _head_body(g_ref, sums_ref, woh_ref, wob_ref, bo_ref, out_ref):
    sums = sums_ref[...]
    s = sums[:, 0:1]
    e = sums[:, 1:4]
    bary = jnp.where(s > 0, e / jnp.where(s > 0, s, 1.0), 0.0)
    out_ref[...] = (
        jnp.dot(g_ref[...], woh_ref[...], preferred_element_type=jnp.float32)
        + jnp.dot(bary, wob_ref[...], preferred_element_type=jnp.float32)
        + bo_ref[...])


def _head(g, sums, woh, wob, bo):
    return pl.pallas_call(
        _head_body,
        out_shape=jax.ShapeDtypeStruct((B, OUT), jnp.float32),
    )(g, sums, woh, wob, bo)


def kernel(x, a, i, W1, b1, W2, b2, g1, be1, g2, be2, Wf, bf, Wa, ba, Wo, bo):
    seg = i.astype(jnp.int32)
    i3d = seg.reshape(NBLK, 1, BLK)
    row = lambda v: v.reshape(1, H)

    xt4 = _xt4(x)
    sums = _bary_sc(xt4, seg)

    g = _main(x, W1, a, row(b1), row(g1), row(be1), W2,
              row(b2), row(g2), row(be2), Wf, row(bf), Wa, row(ba), i3d)

    return _head(g, sums, Wo[:H, :], Wo[H:, :], bo.reshape(1, OUT))


# speedup vs baseline: 1.2047x; 1.0130x over previous
import functools

import jax
import jax.numpy as jnp
from jax import lax
from jax.experimental import pallas as pl
from jax.experimental.pallas import tpu as pltpu
from jax.experimental.pallas import tpu_sc as plsc

N = 10000
F = 128
H = 64
B = 64
OUT = 3
EPS = 1e-3
BLK = 400
HBLK = BLK // 2
NBLK = N // BLK
LANE = 16


def _layernorm(h, gamma, beta):
    mu = jnp.mean(h, axis=-1, keepdims=True)
    d = h - mu
    var = jnp.mean(d * d, axis=-1, keepdims=True)
    return d * lax.rsqrt(var + EPS) * gamma + beta


def _xt4_body(x_ref, xt4_ref):
    r = lax.broadcasted_iota(jnp.int32, (8, F), 0)
    f = lax.broadcasted_iota(jnp.int32, (8, F), 1)
    tgt = jnp.where(r == 0, 0,
                    jnp.where(r == 1, F - 3,
                              jnp.where(r == 2, F - 2,
                                        jnp.where(r == 3, F - 1, -1))))
    sel = (f == tgt).astype(jnp.float32)
    xt4_ref[...] = lax.dot_general(
        sel, x_ref[...], (((1,), (1,)), ((), ())),
        preferred_element_type=jnp.float32)


def _xt4(x):
    return pl.pallas_call(
        _xt4_body,
        out_shape=jax.ShapeDtypeStruct((8, N), jnp.float32),
    )(x)


def _main_body(x_ref, w1_ref, a0_ref, b1_ref, g1_ref, be1_ref,
               w2_ref, b2_ref, g2_ref, be2_ref, wf_ref, bf_ref, wa_ref,
               ba_ref, i_ref, g_ref, xw_s, h1_s, rhs2_s):
    m = pl.program_id(0)
    mm = m % NBLK

    @pl.when(m == 0)
    def _():
        xw_s[...] = jnp.dot(x_ref[...], w1_ref[...],
                            preferred_element_type=jnp.float32)
        g_ref[...] = jnp.zeros_like(g_ref)

    @pl.when(m < NBLK)
    def _():
        rows = pl.ds(mm * BLK, BLK)
        t = jnp.dot(a0_ref[...], xw_s[...],
                    preferred_element_type=jnp.float32)
        h = jnp.maximum(t + b1_ref[...], 0.0)
        hn = _layernorm(h, g1_ref[...], be1_ref[...])
        h1_s[rows, :] = hn
        rhs2_s[rows, :] = jnp.dot(hn, w2_ref[...],
                                  preferred_element_type=jnp.float32)

    @pl.when(m >= NBLK)
    def _():
        rows = pl.ds(mm * BLK, BLK)
        t = jnp.dot(a0_ref[...], rhs2_s[...],
                    preferred_element_type=jnp.float32)
        h = jnp.maximum(t + b2_ref[...], 0.0)
        h = _layernorm(h, g2_ref[...], be2_ref[...]) + h1_s[rows, :]
        feat = jnp.dot(h, wf_ref[...],
                       preferred_element_type=jnp.float32) + bf_ref[...]
        attn = jax.nn.sigmoid(
            jnp.dot(h, wa_ref[...], preferred_element_type=jnp.float32)
            + ba_ref[...])
        contrib = feat * attn
        seg = i_ref[pl.ds(mm, 1), :]
        onehot = (lax.broadcasted_iota(jnp.int32, (B, BLK), 0) == seg)
        part = jnp.dot(onehot.astype(jnp.float32), contrib,
                       preferred_element_type=jnp.float32)
        g_ref[...] += part


def _main(x, w1, a, b1, g1, be1, w2, b2, g2, be2, wf, bf, wa, ba, i3d):
    vec = lambda: pl.BlockSpec((1, H), lambda m: (0, 0))
    mat = lambda: pl.BlockSpec((H, H), lambda m: (0, 0))
    return pl.pallas_call(
        _main_body,
        grid=(2 * NBLK,),
        in_specs=[
            pl.BlockSpec((N, F), lambda m: (0, 0)),
            pl.BlockSpec((F, H), lambda m: (0, 0)),
            pl.BlockSpec((BLK, N), lambda m: (m % NBLK, 0)),
            vec(), vec(), vec(),
            mat(),
            vec(), vec(), vec(),
            mat(), vec(), mat(), vec(),
            pl.BlockSpec((NBLK, BLK), lambda m: (0, 0)),
        ],
        out_specs=pl.BlockSpec((B, H), lambda m: (0, 0)),
        out_shape=jax.ShapeDtypeStruct((B, H), jnp.float32),
        scratch_shapes=[
            pltpu.VMEM((N, H), jnp.float32),
            pltpu.VMEM((N, H), jnp.float32),
            pltpu.VMEM((N, H), jnp.float32),
        ],
        compiler_params=pltpu.CompilerParams(
            dimension_semantics=("arbitrary",)),
    )(x, w1, a, b1, g1, be1, w2, b2, g2, be2, wf, bf, wa, ba, i3d)


def _bary_sc(xt4, iseg):
    mesh = plsc.VectorSubcoreMesh(core_axis_name="c", subcore_axis_name="s")

    @functools.partial(
        pl.kernel,
        mesh=mesh,
        out_type=jax.ShapeDtypeStruct((B, LANE), jnp.float32),
        scratch_types=[
            pltpu.VMEM((4, N), jnp.float32),
            pltpu.VMEM((N,), jnp.int32),
            pltpu.VMEM((2, LANE), jnp.float32),
        ],
    )
    def bary_kernel(xt4_hbm, iseg_hbm, out_hbm, xt_v, iseg_v, out_v):
        wid = lax.axis_index("s") * 2 + lax.axis_index("c")
        pltpu.sync_copy(xt4_hbm.at[pl.ds(0, 4)], xt_v)
        pltpu.sync_copy(iseg_hbm, iseg_v)
        sig_v = xt_v.at[0]
        xs_v = xt_v.at[1]
        ys_v = xt_v.at[2]
        zs_v = xt_v.at[3]
        b0 = wid * 2
        il = lax.broadcasted_iota(jnp.int32, (LANE,), 0)
        dnums = lax.GatherDimensionNumbers(
            offset_dims=(), collapsed_slice_dims=(0,),
            start_index_map=(0,))

        NCH = N // LANE

        def allsum(v):
            for k in (8, 4, 2, 1):
                perm = lax.gather(
                    v, (il ^ k)[:, None], dnums, slice_sizes=(1,),
                    mode=lax.GatherScatterMode.PROMISE_IN_BOUNDS)
                v = v + perm
            return v

        def find_ge(b):
            def step(_, c):
                lo, hi = c
                mid = lax.div(lo + hi, 2)
                v = iseg_v[pl.ds(mid * LANE, LANE)]
                lt = v[LANE - 1] < b
                return (jnp.where(lt, mid + 1, lo), jnp.where(lt, hi, mid))

            ch = lax.fori_loop(0, 10, step, (0, NCH))[0]
            ch = jnp.minimum(ch, NCH - 1)
            v = iseg_v[pl.ds(ch * LANE, LANE)]
            cnt = allsum(jnp.where(v < b, 1.0, 0.0))[0].astype(jnp.int32)
            return ch * LANE + cnt

        s0 = find_ge(b0)
        s1 = find_ge(b0 + 1)
        s2 = find_ge(b0 + 2)

        def seg_sums(start, end):
            c0 = start // LANE
            c1 = (end + LANE - 1) // LANE

            def body(c, acc):
                accs, accx, accy, accz = acc
                off = c * LANE
                lane = off + il
                msk = (lane >= start) & (lane < end)
                sg = jnp.maximum(sig_v[pl.ds(off, LANE)], 0.0)
                w = jnp.where(msk, 1.0 + sg, 0.0)
                return (accs + w,
                        accx + w * xs_v[pl.ds(off, LANE)],
                        accy + w * ys_v[pl.ds(off, LANE)],
                        accz + w * zs_v[pl.ds(off, LANE)])

            z16 = jnp.zeros((LANE,), jnp.float32)
            accs, accx, accy, accz = lax.fori_loop(
                c0, c1, body, (z16, z16, z16, z16))
            return (allsum(accs), allsum(accx),
                    allsum(accy), allsum(accz))

        for j, (lo, hi) in enumerate(((s0, s1), (s1, s2))):
            ssum, ex, ey, ez = seg_sums(lo, hi)
            row = jnp.where(
                il == 0, ssum,
                jnp.where(il == 1, ex,
                          jnp.where(il == 2, ey,
                                    jnp.where(il == 3, ez,
                                              jnp.zeros((LANE,),
                                                        jnp.float32)))))
            out_v[j] = row
        pltpu.sync_copy(out_v, out_hbm.at[pl.ds(b0, 2)])

    return bary_kernel(xt4, iseg)


def _head_body(g_ref, sums_ref, woh_ref, wob_ref, bo_ref, out_ref):
    sums = sums_ref[...]
    s = sums[:, 0:1]
    e = sums[:, 1:4]
    bary = jnp.where(s > 0, e / jnp.where(s > 0, s, 1.0), 0.0)
    out_ref[...] = (
        jnp.dot(g_ref[...], woh_ref[...], preferred_element_type=jnp.float32)
        + jnp.dot(bary, wob_ref[...], preferred_element_type=jnp.float32)
        + bo_ref[...])


def _head(g, sums, woh, wob, bo):
    return pl.pallas_call(
        _head_body,
        out_shape=jax.ShapeDtypeStruct((B, OUT), jnp.float32),
    )(g, sums, woh, wob, bo)


def kernel(x, a, i, W1, b1, W2, b2, g1, be1, g2, be2, Wf, bf, Wa, ba, Wo, bo):
    seg = i.astype(jnp.int32)
    i2d = seg.reshape(NBLK, BLK)
    row = lambda v: v.reshape(1, H)

    xt4 = _xt4(x)
    sums = _bary_sc(xt4, seg)

    g = _main(x, W1, a, row(b1), row(g1), row(be1), W2,
              row(b2), row(g2), row(be2), Wf, row(bf), Wa, row(ba), i2d)

    return _head(g, sums, Wo[:H, :], Wo[H:, :], bo.reshape(1, OUT))
